# baseline (speedup 1.0000x reference)
import functools

import jax
import jax.numpy as jnp
from jax import lax
from jax.experimental import pallas as pl
from jax.experimental.pallas import tpu as pltpu
from jax.experimental.pallas import tpu_sc as plsc

NSEM = 16


@functools.lru_cache(maxsize=None)
def _build(batch: int, seq: int, d_model: int):
    info = plsc.get_sparse_core_info()
    nc, ns = info.num_cores, info.num_subcores
    nw = nc * ns
    n_tokens = batch * seq
    assert n_tokens % (nw * NSEM) == 0
    bpw = n_tokens // nw
    wpr = seq // bpw
    mesh = plsc.VectorSubcoreMesh(core_axis_name="c", subcore_axis_name="s")

    @functools.partial(
        pl.kernel,
        mesh=mesh,
        out_type=jax.ShapeDtypeStruct((n_tokens, d_model), jnp.float32),
        scratch_types=(
            [pltpu.VMEM((bpw,), jnp.int32)]
            + [pltpu.SemaphoreType.DMA for _ in range(NSEM)]
        ),
    )
    def emb(table_hbm, idx_hbm, out_hbm, idx_sm, *sems):
        wid = lax.axis_index("s") * nc + lax.axis_index("c")
        base = wid * bpw
        pltpu.sync_copy(
            idx_hbm.at[wid // wpr, pl.ds((wid % wpr) * bpw, bpw)], idx_sm)

        def wait(k):
            pltpu.make_async_copy(
                table_hbm.at[pl.ds(0, 1)], out_hbm.at[pl.ds(0, 1)],
                sems[k]).wait()

        def body(o, _):
            i0 = pl.multiple_of(o * NSEM, NSEM)
            rows = idx_sm[pl.ds(i0, NSEM)]
            for k in range(NSEM):
                @pl.when(o > 0)
                def _():
                    wait(k)

                pltpu.async_copy(
                    table_hbm.at[pl.ds(rows[k], 1)],
                    out_hbm.at[pl.ds(base + i0 + k, 1)], sems[k])
            return 0

        lax.fori_loop(0, bpw // NSEM, body, 0)
        for k in range(NSEM):
            wait(k)

    return emb


def kernel(token_ids, weight):
    batch, seq = token_ids.shape
    out = _build(batch, seq, weight.shape[1])(
        weight, token_ids.astype(jnp.int32))
    return out.reshape(batch, seq, weight.shape[1])


# speedup vs baseline: 31.3710x; 31.3710x over previous
import functools

import jax
import jax.numpy as jnp
from jax import lax
from jax.experimental import pallas as pl
from jax.experimental.pallas import tpu as pltpu
from jax.experimental.pallas import tpu_sc as plsc

CHUNK = 8
NBUF = 8


@functools.lru_cache(maxsize=None)
def _build(batch: int, seq: int, d_model: int):
    info = plsc.get_sparse_core_info()
    nc, ns = info.num_cores, info.num_subcores
    nw = nc * ns
    n_tokens = batch * seq
    assert n_tokens % (nw * CHUNK) == 0 and seq % CHUNK == 0
    bpw = n_tokens // nw
    wpr = seq // bpw
    nchunks = bpw // CHUNK
    assert nchunks % NBUF == 0
    mesh = plsc.VectorSubcoreMesh(core_axis_name="c", subcore_axis_name="s")

    @functools.partial(
        pl.kernel,
        mesh=mesh,
        out_type=jax.ShapeDtypeStruct((n_tokens, d_model), jnp.float32),
        scratch_types=(
            [pltpu.VMEM((bpw,), jnp.int32)]
            + [pltpu.VMEM((CHUNK, d_model), jnp.float32) for _ in range(NBUF)]
            + [pltpu.SemaphoreType.DMA for _ in range(2 * NBUF)]
        ),
    )
    def emb(table_hbm, idx_hbm, out_hbm, idx_v, *rest):
        bufs = rest[:NBUF]
        gsems = rest[NBUF:2 * NBUF]
        ssems = rest[2 * NBUF:]
        wid = lax.axis_index("s") * nc + lax.axis_index("c")
        base = wid * bpw
        pltpu.sync_copy(
            idx_hbm.at[wid // wpr, pl.ds((wid % wpr) * bpw, bpw)], idx_v)

        def start_gather(g, b):
            off = pl.multiple_of(g * CHUNK, CHUNK)
            pltpu.async_copy(
                table_hbm.at[idx_v.at[pl.ds(off, CHUNK)]], bufs[b], gsems[b])

        def wait_gather(b):
            pltpu.make_async_copy(
                table_hbm.at[idx_v.at[pl.ds(0, CHUNK)]], bufs[b],
                gsems[b]).wait()

        def start_store(g, b):
            off = pl.multiple_of(base + g * CHUNK, CHUNK)
            pltpu.async_copy(bufs[b], out_hbm.at[pl.ds(off, CHUNK)], ssems[b])

        def wait_store(b):
            pltpu.make_async_copy(
                bufs[b], out_hbm.at[pl.ds(0, CHUNK)], ssems[b]).wait()

        for b in range(NBUF):
            start_gather(b, b)

        def body(go, _):
            g0 = go * NBUF
            for b in range(NBUF):
                g = g0 + b
                wait_gather(b)
                start_store(g, b)
                ng = g + NBUF

                @pl.when(ng < nchunks)
                def _():
                    wait_store(b)
                    start_gather(ng, b)

            return 0

        lax.fori_loop(0, nchunks // NBUF, body, 0)
        for b in range(NBUF):
            wait_store(b)

    return emb


def kernel(token_ids, weight):
    batch, seq = token_ids.shape
    out = _build(batch, seq, weight.shape[1])(
        weight, token_ids.astype(jnp.int32))
    return out.reshape(batch, seq, weight.shape[1])
